# baseline (speedup 1.0000x reference)
import functools

import jax
import jax.numpy as jnp
from jax import lax
from jax.experimental import pallas as pl
from jax.experimental.pallas import tpu as pltpu
from jax.experimental.pallas import tpu_sc as plsc

N = 10000
E = 160000
D = 256
H = 128
NC = 2
NS = 16
BATCH = 128
EPAD = 163840
NB3 = EPAD // (NS * BATCH)
NB1 = EPAD // (NC * NS * BATCH)
HB3 = NB3 // 2
HB1 = NB1 // 2
R = 10240
STRIPE = R // NS
RB = 2000


def _k1_body(dst_hbm, ones_hbm, zeros_hbm, out_hbm, idx_v, ones_v, acc_sh):
    c = lax.axis_index("c")
    s = lax.axis_index("s")
    w = s * NC + c
    pltpu.sync_copy(zeros_hbm, acc_sh.at[pl.ds(s * STRIPE, STRIPE)])
    pltpu.sync_copy(ones_hbm, ones_v)
    plsc.subcore_barrier()

    for h in range(2):
        pltpu.sync_copy(dst_hbm.at[h].at[w], idx_v)

        @pl.loop(0, HB1)
        def _(b):
            pltpu.sync_copy(ones_v, acc_sh.at[idx_v.at[b]], add=True)

    plsc.subcore_barrier()
    pltpu.sync_copy(
        acc_sh.at[pl.ds(s * STRIPE, STRIPE)],
        out_hbm.at[c].at[pl.ds(s * STRIPE, STRIPE)],
    )


def _k3_body(zl_hbm, zr_hbm, src_hbm, dst_hbm, zeros_hbm, out_hbm,
             src_v, dst_v, buf0_v, buf1_v, acc_sh, sem0, sem1):
    c = lax.axis_index("c")
    s = lax.axis_index("s")
    pltpu.sync_copy(zeros_hbm, acc_sh.at[pl.ds(s * STRIPE, STRIPE)])
    plsc.subcore_barrier()

    def edge_loop(z_hbm):
        for h in range(2):
            pltpu.sync_copy(src_hbm.at[h].at[s], src_v)
            pltpu.sync_copy(dst_hbm.at[h].at[s], dst_v)
            pltpu.async_copy(z_hbm.at[src_v.at[0]], buf0_v, sem0)

            @pl.loop(0, HB3, step=2)
            def _(b):
                pltpu.make_async_copy(z_hbm.at[src_v.at[b]], buf0_v, sem0).wait()
                pltpu.async_copy(z_hbm.at[src_v.at[b + 1]], buf1_v, sem1)
                pltpu.sync_copy(buf0_v, acc_sh.at[dst_v.at[b]], add=True)
                pltpu.make_async_copy(z_hbm.at[src_v.at[b + 1]], buf1_v, sem1).wait()

                @pl.when(b + 2 < HB3)
                def _():
                    pltpu.async_copy(z_hbm.at[src_v.at[b + 2]], buf0_v, sem0)

                pltpu.sync_copy(buf1_v, acc_sh.at[dst_v.at[b + 1]], add=True)

    @pl.when(c == 0)
    def _():
        edge_loop(zl_hbm)

    @pl.when(c == 1)
    def _():
        edge_loop(zr_hbm)

    plsc.subcore_barrier()
    pltpu.sync_copy(
        acc_sh.at[pl.ds(s * STRIPE, STRIPE)],
        out_hbm.at[c].at[pl.ds(s * STRIPE, STRIPE)],
    )


def _k2_body(x_ref, w_ref, deg_ref, zl_ref, zr_ref):
    deg = deg_ref[:, 0:1] + deg_ref[:, 1:2]
    norm = lax.rsqrt(jnp.maximum(deg, 1.0))
    z = jnp.dot(x_ref[...], w_ref[...], preferred_element_type=jnp.float32) * norm
    zl_ref[...] = z[:, :H]
    zr_ref[...] = z[:, H:]


def _k4_body(agg_ref, deg_ref, b_ref, out_ref):
    deg = deg_ref[:, 0:1] + deg_ref[:, 1:2]
    norm = lax.rsqrt(jnp.maximum(deg, 1.0))
    out_ref[:, :H] = agg_ref[0] * norm + b_ref[0:1, :H]
    out_ref[:, H:] = agg_ref[1] * norm + b_ref[0:1, H:]


@jax.jit
def kernel(x, edge_index, W, b):
    mesh = plsc.VectorSubcoreMesh(
        core_axis_name="c", subcore_axis_name="s", num_cores=NC, num_subcores=NS
    )
    f32 = jnp.float32

    src = edge_index[0]
    dst = edge_index[1]
    pad = EPAD - E
    srcp = jnp.concatenate([src, jnp.zeros((pad,), jnp.int32)])
    dstp = jnp.concatenate([dst, jnp.full((pad,), N, jnp.int32)])
    dst1 = dstp.reshape(2, NC * NS, HB1, BATCH)
    src3 = srcp.reshape(2, NS, HB3, BATCH)
    dst3 = dstp.reshape(2, NS, HB3, BATCH)

    zeros1 = jnp.zeros((STRIPE,), f32)
    zerosS = jnp.zeros((STRIPE, H), f32)

    k1 = pl.kernel(
        _k1_body,
        out_type=jax.ShapeDtypeStruct((NC, R), f32),
        mesh=mesh,
        scratch_types=[
            pltpu.VMEM((HB1, BATCH), jnp.int32),
            pltpu.VMEM((BATCH,), f32),
            pltpu.VMEM_SHARED((R,), f32),
        ],
    )
    degacc = k1(dst1, jnp.ones((BATCH,), f32), zeros1)
    degT = degacc[:, :N].T

    k2 = pl.pallas_call(
        _k2_body,
        grid=(N // RB,),
        in_specs=[
            pl.BlockSpec((RB, D), lambda i: (i, 0)),
            pl.BlockSpec((D, D), lambda i: (0, 0)),
            pl.BlockSpec((RB, NC), lambda i: (i, 0)),
        ],
        out_specs=[
            pl.BlockSpec((RB, H), lambda i: (i, 0)),
            pl.BlockSpec((RB, H), lambda i: (i, 0)),
        ],
        out_shape=[
            jax.ShapeDtypeStruct((N, H), f32),
            jax.ShapeDtypeStruct((N, H), f32),
        ],
    )
    zl, zr = k2(x, W, degT)

    k3 = pl.kernel(
        _k3_body,
        out_type=jax.ShapeDtypeStruct((NC, R, H), f32),
        mesh=mesh,
        scratch_types=[
            pltpu.VMEM((HB3, BATCH), jnp.int32),
            pltpu.VMEM((HB3, BATCH), jnp.int32),
            pltpu.VMEM((BATCH, H), f32),
            pltpu.VMEM((BATCH, H), f32),
            pltpu.VMEM_SHARED((R, H), f32),
            pltpu.SemaphoreType.DMA,
            pltpu.SemaphoreType.DMA,
        ],
    )
    agg = k3(zl, zr, src3, dst3, zerosS)

    k4 = pl.pallas_call(
        _k4_body,
        grid=(N // RB,),
        in_specs=[
            pl.BlockSpec((NC, RB, H), lambda i: (0, i, 0)),
            pl.BlockSpec((RB, NC), lambda i: (i, 0)),
            pl.BlockSpec((1, D), lambda i: (0, 0)),
        ],
        out_specs=pl.BlockSpec((RB, D), lambda i: (i, 0)),
        out_shape=jax.ShapeDtypeStruct((N, D), f32),
    )
    return k4(agg[:, :N], degT, b.reshape(1, D))


# speedup vs baseline: 1.0192x; 1.0192x over previous
import functools

import jax
import jax.numpy as jnp
from jax import lax
from jax.experimental import pallas as pl
from jax.experimental.pallas import tpu as pltpu
from jax.experimental.pallas import tpu_sc as plsc

N = 10000
E = 160000
D = 256
H = 128
NC = 2
NS = 16
BATCH = 128
EPAD = 163840
NB3 = EPAD // (NS * BATCH)
NB1 = EPAD // (NC * NS * BATCH)
HB3 = NB3 // 2
HB1 = NB1 // 2
R = 10240
STRIPE = R // NS
RB = 2000


def _k1_body(dst_hbm, ones_hbm, zeros_hbm, out_hbm, idx_v, ones_v, acc_sh):
    c = lax.axis_index("c")
    s = lax.axis_index("s")
    w = s * NC + c
    pltpu.sync_copy(zeros_hbm, acc_sh.at[pl.ds(s * STRIPE, STRIPE)])
    pltpu.sync_copy(ones_hbm, ones_v)
    plsc.subcore_barrier()

    for h in range(2):
        pltpu.sync_copy(dst_hbm.at[h].at[w], idx_v)

        @pl.loop(0, HB1)
        def _(b):
            pltpu.sync_copy(ones_v, acc_sh.at[idx_v.at[b]], add=True)

    plsc.subcore_barrier()
    pltpu.sync_copy(
        acc_sh.at[pl.ds(s * STRIPE, STRIPE)],
        out_hbm.at[c].at[pl.ds(s * STRIPE, STRIPE)],
    )


def _k3_body(zl_hbm, zr_hbm, src_hbm, dst_hbm, zeros_hbm, out_hbm,
             src_v, dst_v, buf0_v, buf1_v, acc_sh, sem0, sem1):
    c = lax.axis_index("c")
    s = lax.axis_index("s")
    pltpu.sync_copy(zeros_hbm, acc_sh.at[pl.ds(s * STRIPE, STRIPE)])
    plsc.subcore_barrier()

    def edge_loop(z_hbm):
        for h in range(2):
            pltpu.sync_copy(src_hbm.at[h].at[s], src_v)
            pltpu.sync_copy(dst_hbm.at[h].at[s], dst_v)
            pltpu.async_copy(z_hbm.at[src_v.at[0]], buf0_v, sem0)

            @pl.loop(0, HB3, step=2)
            def _(b):
                pltpu.make_async_copy(z_hbm.at[src_v.at[b]], buf0_v, sem0).wait()
                pltpu.async_copy(z_hbm.at[src_v.at[b + 1]], buf1_v, sem1)
                pltpu.sync_copy(buf0_v, acc_sh.at[dst_v.at[b]], add=True)
                pltpu.make_async_copy(z_hbm.at[src_v.at[b + 1]], buf1_v, sem1).wait()

                @pl.when(b + 2 < HB3)
                def _():
                    pltpu.async_copy(z_hbm.at[src_v.at[b + 2]], buf0_v, sem0)

                pltpu.sync_copy(buf1_v, acc_sh.at[dst_v.at[b + 1]], add=True)

    @pl.when(c == 0)
    def _():
        edge_loop(zl_hbm)

    @pl.when(c == 1)
    def _():
        edge_loop(zr_hbm)

    plsc.subcore_barrier()
    pltpu.sync_copy(
        acc_sh.at[pl.ds(s * STRIPE, STRIPE)],
        out_hbm.at[c].at[pl.ds(s * STRIPE, STRIPE)],
    )


def _k2_body(x_ref, w_ref, deg_ref, zl_ref, zr_ref):
    deg = deg_ref[:, 0:1] + deg_ref[:, 1:2]
    norm = lax.rsqrt(jnp.maximum(deg, 1.0))
    z = jnp.dot(x_ref[...], w_ref[...], preferred_element_type=jnp.float32) * norm
    zl_ref[...] = z[:, :H]
    zr_ref[...] = z[:, H:]


def _k4_body(agg_ref, deg_ref, b_ref, out_ref):
    deg = deg_ref[:, 0:1] + deg_ref[:, 1:2]
    norm = lax.rsqrt(jnp.maximum(deg, 1.0))
    out_ref[:, :H] = agg_ref[0] * norm + b_ref[0:1, :H]
    out_ref[:, H:] = agg_ref[1] * norm + b_ref[0:1, H:]


@jax.jit
def kernel(x, edge_index, W, b):
    mesh = plsc.VectorSubcoreMesh(
        core_axis_name="c", subcore_axis_name="s", num_cores=NC, num_subcores=NS
    )
    f32 = jnp.float32

    src = edge_index[0]
    dst = edge_index[1]
    pad = EPAD - E
    srcp = jnp.concatenate([src, jnp.zeros((pad,), jnp.int32)])
    dstp = jnp.concatenate([dst, jnp.full((pad,), N, jnp.int32)])
    dst1 = dstp.reshape(2, NC * NS, HB1, BATCH)
    src3 = srcp.reshape(2, NS, HB3, BATCH)
    dst3 = dstp.reshape(2, NS, HB3, BATCH)

    zeros1 = jnp.zeros((STRIPE,), f32)
    zerosS = jnp.zeros((STRIPE, H), f32)

    k1 = pl.kernel(
        _k1_body,
        out_type=jax.ShapeDtypeStruct((NC, R), f32),
        mesh=mesh,
        scratch_types=[
            pltpu.VMEM((HB1, BATCH), jnp.int32),
            pltpu.VMEM((BATCH,), f32),
            pltpu.VMEM_SHARED((R,), f32),
        ],
    )
    degacc = k1(dst1, jnp.ones((BATCH,), f32), zeros1)
    degT = degacc[:, :N].T

    k2 = pl.pallas_call(
        _k2_body,
        grid=(N // RB,),
        in_specs=[
            pl.BlockSpec((RB, D), lambda i: (i, 0)),
            pl.BlockSpec((D, D), lambda i: (0, 0)),
            pl.BlockSpec((RB, NC), lambda i: (i, 0)),
        ],
        out_specs=[
            pl.BlockSpec((RB, H), lambda i: (i, 0)),
            pl.BlockSpec((RB, H), lambda i: (i, 0)),
        ],
        out_shape=[
            jax.ShapeDtypeStruct((N, H), f32),
            jax.ShapeDtypeStruct((N, H), f32),
        ],
    )
    zl, zr = k2(x, W, degT)

    k3 = pl.kernel(
        _k3_body,
        out_type=jax.ShapeDtypeStruct((NC, R, H), f32),
        mesh=mesh,
        scratch_types=[
            pltpu.VMEM((HB3, BATCH), jnp.int32),
            pltpu.VMEM((HB3, BATCH), jnp.int32),
            pltpu.VMEM((BATCH, H), f32),
            pltpu.VMEM((BATCH, H), f32),
            pltpu.VMEM_SHARED((R, H), f32),
            pltpu.SemaphoreType.DMA,
            pltpu.SemaphoreType.DMA,
        ],
    )
    agg = k3(zl, zr, src3, dst3, zerosS)

    k4 = pl.pallas_call(
        _k4_body,
        grid=(N // RB,),
        in_specs=[
            pl.BlockSpec((NC, RB, H), lambda i: (0, i, 0)),
            pl.BlockSpec((RB, NC), lambda i: (i, 0)),
            pl.BlockSpec((1, D), lambda i: (0, 0)),
        ],
        out_specs=pl.BlockSpec((RB, D), lambda i: (i, 0)),
        out_shape=jax.ShapeDtypeStruct((N, D), f32),
    )
    return k4(agg, degT, b.reshape(1, D))


# speedup vs baseline: 1.0196x; 1.0004x over previous
import functools

import jax
import jax.numpy as jnp
from jax import lax
from jax.experimental import pallas as pl
from jax.experimental.pallas import tpu as pltpu
from jax.experimental.pallas import tpu_sc as plsc

N = 10000
E = 160000
D = 256
H = 128
NC = 2
NS = 16
BATCH = 128
EPAD = 163840
NB3 = EPAD // (NS * BATCH)
NB1 = EPAD // (NC * NS * BATCH)
HB3 = NB3 // 2
HB1 = NB1 // 2
R = 10240
STRIPE = R // NS
RB = 2000


def _k1_body(dst_hbm, ones_hbm, zeros_hbm, out_hbm, idx_v, ones_v, acc_sh):
    c = lax.axis_index("c")
    s = lax.axis_index("s")
    w = s * NC + c
    pltpu.sync_copy(zeros_hbm, acc_sh.at[pl.ds(s * STRIPE, STRIPE)])
    pltpu.sync_copy(ones_hbm, ones_v)
    plsc.subcore_barrier()

    for h in range(2):
        pltpu.sync_copy(dst_hbm.at[h].at[w], idx_v)

        @pl.loop(0, HB1)
        def _(b):
            pltpu.sync_copy(ones_v, acc_sh.at[idx_v.at[b]], add=True)

    plsc.subcore_barrier()
    pltpu.sync_copy(
        acc_sh.at[pl.ds(s * STRIPE, STRIPE)],
        out_hbm.at[c].at[pl.ds(s * STRIPE, STRIPE)],
    )


def _k3_body(zl_hbm, zr_hbm, src_hbm, dst_hbm, zeros_hbm, out_hbm,
             src_v, dst_v, buf0_v, buf1_v, acc_sh, semg0, semg1, sems0, sems1):
    c = lax.axis_index("c")
    s = lax.axis_index("s")
    pltpu.sync_copy(zeros_hbm, acc_sh.at[pl.ds(s * STRIPE, STRIPE)])
    plsc.subcore_barrier()

    def edge_loop(z_hbm):
        for h in range(2):
            pltpu.sync_copy(src_hbm.at[h].at[s], src_v)
            pltpu.sync_copy(dst_hbm.at[h].at[s], dst_v)
            pltpu.async_copy(z_hbm.at[src_v.at[0]], buf0_v, semg0)

            @pl.loop(0, HB3, step=2)
            def _(b):
                pltpu.make_async_copy(z_hbm.at[src_v.at[b]], buf0_v, semg0).wait()
                pltpu.async_copy(buf0_v, acc_sh.at[dst_v.at[b]], sems0, add=True)

                @pl.when(b > 0)
                def _():
                    pltpu.make_async_copy(
                        buf1_v, acc_sh.at[dst_v.at[b - 1]], sems1).wait()

                pltpu.async_copy(z_hbm.at[src_v.at[b + 1]], buf1_v, semg1)
                pltpu.make_async_copy(z_hbm.at[src_v.at[b + 1]], buf1_v, semg1).wait()
                pltpu.async_copy(buf1_v, acc_sh.at[dst_v.at[b + 1]], sems1, add=True)
                pltpu.make_async_copy(buf0_v, acc_sh.at[dst_v.at[b]], sems0).wait()

                @pl.when(b + 2 < HB3)
                def _():
                    pltpu.async_copy(z_hbm.at[src_v.at[b + 2]], buf0_v, semg0)

            pltpu.make_async_copy(
                buf1_v, acc_sh.at[dst_v.at[HB3 - 1]], sems1).wait()

    @pl.when(c == 0)
    def _():
        edge_loop(zl_hbm)

    @pl.when(c == 1)
    def _():
        edge_loop(zr_hbm)

    plsc.subcore_barrier()
    pltpu.sync_copy(
        acc_sh.at[pl.ds(s * STRIPE, STRIPE)],
        out_hbm.at[c].at[pl.ds(s * STRIPE, STRIPE)],
    )


def _k2_body(x_ref, w_ref, deg_ref, zl_ref, zr_ref):
    deg = deg_ref[:, 0:1] + deg_ref[:, 1:2]
    norm = lax.rsqrt(jnp.maximum(deg, 1.0))
    z = jnp.dot(x_ref[...], w_ref[...], preferred_element_type=jnp.float32) * norm
    zl_ref[...] = z[:, :H]
    zr_ref[...] = z[:, H:]


def _k4_body(agg_ref, deg_ref, b_ref, out_ref):
    deg = deg_ref[:, 0:1] + deg_ref[:, 1:2]
    norm = lax.rsqrt(jnp.maximum(deg, 1.0))
    out_ref[:, :H] = agg_ref[0] * norm + b_ref[0:1, :H]
    out_ref[:, H:] = agg_ref[1] * norm + b_ref[0:1, H:]


@jax.jit
def kernel(x, edge_index, W, b):
    mesh = plsc.VectorSubcoreMesh(
        core_axis_name="c", subcore_axis_name="s", num_cores=NC, num_subcores=NS
    )
    f32 = jnp.float32

    src = edge_index[0]
    dst = edge_index[1]
    pad = EPAD - E
    srcp = jnp.concatenate([src, jnp.zeros((pad,), jnp.int32)])
    dstp = jnp.concatenate([dst, jnp.full((pad,), N, jnp.int32)])
    dst1 = dstp.reshape(2, NC * NS, HB1, BATCH)
    src3 = srcp.reshape(2, NS, HB3, BATCH)
    dst3 = dstp.reshape(2, NS, HB3, BATCH)

    zeros1 = jnp.zeros((STRIPE,), f32)
    zerosS = jnp.zeros((STRIPE, H), f32)

    k1 = pl.kernel(
        _k1_body,
        out_type=jax.ShapeDtypeStruct((NC, R), f32),
        mesh=mesh,
        scratch_types=[
            pltpu.VMEM((HB1, BATCH), jnp.int32),
            pltpu.VMEM((BATCH,), f32),
            pltpu.VMEM_SHARED((R,), f32),
        ],
    )
    degacc = k1(dst1, jnp.ones((BATCH,), f32), zeros1)
    degT = degacc[:, :N].T

    k2 = pl.pallas_call(
        _k2_body,
        grid=(N // RB,),
        in_specs=[
            pl.BlockSpec((RB, D), lambda i: (i, 0)),
            pl.BlockSpec((D, D), lambda i: (0, 0)),
            pl.BlockSpec((RB, NC), lambda i: (i, 0)),
        ],
        out_specs=[
            pl.BlockSpec((RB, H), lambda i: (i, 0)),
            pl.BlockSpec((RB, H), lambda i: (i, 0)),
        ],
        out_shape=[
            jax.ShapeDtypeStruct((N, H), f32),
            jax.ShapeDtypeStruct((N, H), f32),
        ],
    )
    zl, zr = k2(x, W, degT)

    k3 = pl.kernel(
        _k3_body,
        out_type=jax.ShapeDtypeStruct((NC, R, H), f32),
        mesh=mesh,
        scratch_types=[
            pltpu.VMEM((HB3, BATCH), jnp.int32),
            pltpu.VMEM((HB3, BATCH), jnp.int32),
            pltpu.VMEM((BATCH, H), f32),
            pltpu.VMEM((BATCH, H), f32),
            pltpu.VMEM_SHARED((R, H), f32),
            pltpu.SemaphoreType.DMA,
            pltpu.SemaphoreType.DMA,
            pltpu.SemaphoreType.DMA,
            pltpu.SemaphoreType.DMA,
        ],
    )
    agg = k3(zl, zr, src3, dst3, zerosS)

    k4 = pl.pallas_call(
        _k4_body,
        grid=(N // RB,),
        in_specs=[
            pl.BlockSpec((NC, RB, H), lambda i: (0, i, 0)),
            pl.BlockSpec((RB, NC), lambda i: (i, 0)),
            pl.BlockSpec((1, D), lambda i: (0, 0)),
        ],
        out_specs=pl.BlockSpec((RB, D), lambda i: (i, 0)),
        out_shape=jax.ShapeDtypeStruct((N, D), f32),
    )
    return k4(agg, degT, b.reshape(1, D))
